# baseline (speedup 1.0000x reference)
import functools

import jax
import jax.numpy as jnp
from jax.experimental import pallas as pl
from jax.experimental.pallas import tpu as pltpu

_PX = 512
_PY = 512
_P = _PX * _PY
_B = 2
_N = 100000
_C = 64

_P_BLK = 32768
_N_BLK = 10000
_NUM_P = _P // _P_BLK
_NUM_N = _N // _N_BLK


def _scatter_body(idx_ref, x_ref, out_ref):
    pb = pl.program_id(1)
    nb = pl.program_id(2)
    base = pb * _P_BLK

    @pl.when(nb == 0)
    def _zero():
        out_ref[...] = jnp.zeros_like(out_ref)

    def body(n, carry):
        s = idx_ref[0, 0, 0, n]
        loc = s - base
        valid = jnp.logical_and(loc >= 0, loc < _P_BLK)
        locc = jnp.where(valid, loc, 0)
        w = jnp.where(valid, 1.0, 0.0)
        out_ref[0, pl.ds(locc, 1), :] += x_ref[0, pl.ds(n, 1), :] * w
        return carry

    jax.lax.fori_loop(0, _N_BLK, body, 0, unroll=False)


_T_BLK = 8192


def _transpose_body(in_ref, out_ref):
    out_ref[0, :, :] = in_ref[0, :, :].T


def kernel(x, indices):
    idx4 = indices.reshape(_B, _NUM_N, 1, _N_BLK)
    grid_pc = pl.pallas_call(
        _scatter_body,
        grid=(_B, _NUM_P, _NUM_N),
        in_specs=[
            pl.BlockSpec((1, 1, 1, _N_BLK), lambda b, p, n: (b, n, 0, 0),
                         memory_space=pltpu.SMEM),
            pl.BlockSpec((1, _N_BLK, _C), lambda b, p, n: (b, n, 0)),
        ],
        out_specs=pl.BlockSpec((1, _P_BLK, _C), lambda b, p, n: (b, p, 0)),
        out_shape=jax.ShapeDtypeStruct((_B, _P, _C), jnp.float32),
    )(idx4, x)

    out = pl.pallas_call(
        _transpose_body,
        grid=(_B, _P // _T_BLK),
        in_specs=[pl.BlockSpec((1, _T_BLK, _C), lambda b, t: (b, t, 0))],
        out_specs=pl.BlockSpec((1, _C, _T_BLK), lambda b, t: (b, 0, t)),
        out_shape=jax.ShapeDtypeStruct((_B, _C, _P), jnp.float32),
    )(grid_pc)
    return out.reshape(_B, _C, _PX, _PY)


# speedup vs baseline: 8.7270x; 8.7270x over previous
import dataclasses
import functools

import jax
import jax.numpy as jnp
from jax import lax
from jax.experimental import pallas as pl
from jax.experimental.pallas import tpu as pltpu
from jax.experimental.pallas import tpu_sc as plsc

_PX = 512
_PY = 512
_P = _PX * _PY
_B = 2
_N = 100000
_C = 64
_W = 128

_NC = 2
_NS = 16
_LANES = 16

_NPAD = 100352
_SHARE = _NPAD // _NS
_NVREG = _SHARE // _LANES

_CH = 4096
_HALF = _P // _NC
_NCHUNK = _HALF // _CH

_K = 128
_NBMAX = (_SHARE + _K - 1) // _K + 1
_ZROWS = _CH // _NS

_SENTINEL = 1 << 30


def _sc_body(x_hbm, idx_hbm, zrow_hbm, out_hbm,
             idx_v, cid_v, lidx_v, rows_v, zbuf_v, chunk_sh):
    core = lax.axis_index("c")
    sub = lax.axis_index("s")
    iota = lax.broadcasted_iota(jnp.int32, (_LANES,), 0)
    rowbase = sub * _ZROWS

    pltpu.sync_copy(zrow_hbm, zbuf_v)
    pltpu.sync_copy(zbuf_v, chunk_sh.at[pl.ds(rowbase, _ZROWS)])
    plsc.subcore_barrier()

    for b in range(_B):
        pltpu.sync_copy(idx_hbm.at[b, pl.ds(sub * _SHARE, _SHARE)], idx_v)

        @pl.loop(0, _NCHUNK)
        def _chunk_pass(cc):
            base = core * _HALF + cc * _CH

            def cbody(i, cnt):
                iv = idx_v[pl.ds(i * _LANES, _LANES)]
                m = jnp.logical_and(iv >= base, iv < base + _CH)
                mi = jnp.where(m, jnp.int32(1), jnp.int32(0))
                cs = plsc.cumsum(mi)
                pos = cnt + cs - 1
                pr = pos >> 7
                pc = pos & 127
                pid = sub * _SHARE + i * _LANES + iota
                plsc.store_scatter(cid_v, [pr, pc], pid, mask=m)
                plsc.store_scatter(lidx_v, [pr, pc], iv - base, mask=m)
                return cnt + jnp.sum(mi)

            cnt = lax.fori_loop(0, _NVREG, cbody, jnp.int32(0))

            nb = (cnt + _K - 1) >> 7
            end = nb * _K

            def pbody(v, _):
                p = v * _LANES + iota
                pm = jnp.logical_and(p >= cnt, p < end)
                plsc.store_scatter(lidx_v, [p >> 7, p & 127],
                                   _CH + sub * 16 + (p & 15), mask=pm)
                plsc.store_scatter(cid_v, [p >> 7, p & 127], p & 127, mask=pm)
                return 0

            lax.fori_loop(cnt >> 4, (end + _LANES - 1) >> 4, pbody, 0)

            def gbody(j, _):
                pltpu.sync_copy(x_hbm.at[b].at[cid_v.at[j]], rows_v)
                pltpu.sync_copy(rows_v, chunk_sh.at[lidx_v.at[j]], add=True)
                return 0

            lax.fori_loop(0, nb, gbody, 0)

            plsc.subcore_barrier()
            pltpu.sync_copy(chunk_sh.at[pl.ds(rowbase, _ZROWS)],
                            out_hbm.at[b].at[pl.ds(base + rowbase, _ZROWS)])
            pltpu.sync_copy(zbuf_v, chunk_sh.at[pl.ds(rowbase, _ZROWS)])
            plsc.subcore_barrier()


_T_BLK = 8192


def _transpose_body(in_ref, out_ref):
    out_ref[0, :, :] = in_ref[0, :, 0, :].T


def kernel(x, indices):
    x_pad = jnp.concatenate(
        [x, jnp.zeros((_B, _N, _W - _C), jnp.float32)], axis=2)
    idx_pad = jnp.pad(indices, ((0, 0), (0, _NPAD - _N)),
                      constant_values=_SENTINEL)
    zrow = jnp.zeros((_ZROWS, _W), jnp.float32)

    cp = pltpu.CompilerParams()
    if "needs_layout_passes" in pltpu.CompilerParams.__dataclass_fields__:
        cp = dataclasses.replace(cp, needs_layout_passes=False)
    mesh = plsc.VectorSubcoreMesh(core_axis_name="c", subcore_axis_name="s")
    grid_pw = pl.kernel(
        _sc_body,
        out_type=jax.ShapeDtypeStruct((_B, _P, _W), jnp.float32),
        mesh=mesh,
        scratch_types=[
            pltpu.VMEM((_SHARE,), jnp.int32),
            pltpu.VMEM((_NBMAX, _K), jnp.int32),
            pltpu.VMEM((_NBMAX, _K), jnp.int32),
            pltpu.VMEM((_K, _W), jnp.float32),
            pltpu.VMEM((_ZROWS, _W), jnp.float32),
            pltpu.VMEM_SHARED((_CH + _NS * 16, _W), jnp.float32),
        ],
        compiler_params=cp,
    )(x_pad, idx_pad, zrow)

    out = pl.pallas_call(
        _transpose_body,
        grid=(_B, _P // _T_BLK),
        in_specs=[pl.BlockSpec((1, _T_BLK, 2, _C), lambda b, t: (b, t, 0, 0))],
        out_specs=pl.BlockSpec((1, _C, _T_BLK), lambda b, t: (b, 0, t)),
        out_shape=jax.ShapeDtypeStruct((_B, _C, _P), jnp.float32),
    )(grid_pw.reshape(_B, _P, 2, _C))
    return out.reshape(_B, _C, _PX, _PY)


# speedup vs baseline: 15.7241x; 1.8018x over previous
import dataclasses
import functools

import jax
import jax.numpy as jnp
from jax import lax
from jax.experimental import pallas as pl
from jax.experimental.pallas import tpu as pltpu
from jax.experimental.pallas import tpu_sc as plsc

_PX = 512
_PY = 512
_P = _PX * _PY
_B = 2
_N = 100000
_C = 64
_W = 128

_NC = 2
_NS = 16
_LANES = 16

_NPAD = 100352
_SHARE = _NPAD // _NS
_NVREG = _SHARE // _LANES

_CH = 4096
_HALF = _P // _NC
_NCHUNK = _HALF // _CH

_K = 128
_NBMAX = (_SHARE + _K - 1) // _K + 1
_ZROWS = _CH // _NS

_SENTINEL = 1 << 30


def _sc_body(x_hbm, idx_hbm, zrow_hbm, out_hbm,
             idx_v, cid_v, lidx_v, rows_v, zbuf_v, chunk_sh):
    core = lax.axis_index("c")
    sub = lax.axis_index("s")
    iota = lax.broadcasted_iota(jnp.int32, (_LANES,), 0)
    rowbase = sub * _ZROWS

    pltpu.sync_copy(zrow_hbm, zbuf_v)
    pltpu.sync_copy(zbuf_v, chunk_sh.at[pl.ds(rowbase, _ZROWS)])
    plsc.subcore_barrier()

    for b in range(_B):
        pltpu.sync_copy(idx_hbm.at[b, pl.ds(sub * _SHARE, _SHARE)], idx_v)

        @pl.loop(0, _NCHUNK)
        def _chunk_pass(cc):
            base = core * _HALF + cc * _CH

            def cbody(i, cnt):
                iv = idx_v[pl.ds(i * _LANES, _LANES)]
                m = jnp.logical_and(iv >= base, iv < base + _CH)
                mi = jnp.where(m, jnp.int32(1), jnp.int32(0))
                cs = plsc.cumsum(mi)
                pos = cnt + cs - 1
                pr = pos >> 7
                pc = pos & 127
                pid = sub * _SHARE + i * _LANES + iota
                plsc.store_scatter(cid_v, [pr, pc], pid, mask=m)
                plsc.store_scatter(lidx_v, [pr, pc], iv - base, mask=m)
                return cnt + jnp.sum(mi)

            cnt = lax.fori_loop(0, _NVREG, cbody, jnp.int32(0))

            nb = (cnt + _K - 1) >> 7
            end = nb * _K

            def pbody(v, _):
                p = v * _LANES + iota
                pm = jnp.logical_and(p >= cnt, p < end)
                plsc.store_scatter(lidx_v, [p >> 7, p & 127],
                                   _CH + sub * 16 + (p & 15), mask=pm)
                plsc.store_scatter(cid_v, [p >> 7, p & 127], p & 127, mask=pm)
                return 0

            lax.fori_loop(cnt >> 4, (end + _LANES - 1) >> 4, pbody, 0)

            def gbody(j, _):
                pltpu.sync_copy(x_hbm.at[b].at[cid_v.at[j]], rows_v)
                pltpu.sync_copy(rows_v, chunk_sh.at[lidx_v.at[j]], add=True)
                return 0

            lax.fori_loop(0, nb, gbody, 0)

            plsc.subcore_barrier()
            pltpu.sync_copy(chunk_sh.at[pl.ds(rowbase, _ZROWS)],
                            out_hbm.at[b].at[pl.ds(base + rowbase, _ZROWS)])
            pltpu.sync_copy(zbuf_v, chunk_sh.at[pl.ds(rowbase, _ZROWS)])
            plsc.subcore_barrier()


_T_BLK = 8192
_PAD_BLK = 10000


def _transpose_body(in_ref, out_ref):
    out_ref[0, :, :] = in_ref[0, :, 0:_C].T


def _pad_body(x_ref, out_ref):
    out_ref[0, :, 0:_C] = x_ref[0]
    out_ref[0, :, _C:_W] = jnp.zeros((_PAD_BLK, _W - _C), jnp.float32)


def kernel(x, indices):
    x_pad = pl.pallas_call(
        _pad_body,
        grid=(_B, _N // _PAD_BLK),
        in_specs=[pl.BlockSpec((1, _PAD_BLK, _C), lambda b, n: (b, n, 0))],
        out_specs=pl.BlockSpec((1, _PAD_BLK, _W), lambda b, n: (b, n, 0)),
        out_shape=jax.ShapeDtypeStruct((_B, _N, _W), jnp.float32),
    )(x)
    idx_pad = jnp.pad(indices, ((0, 0), (0, _NPAD - _N)),
                      constant_values=_SENTINEL)
    zrow = jnp.zeros((_ZROWS, _W), jnp.float32)

    cp = pltpu.CompilerParams()
    if "needs_layout_passes" in pltpu.CompilerParams.__dataclass_fields__:
        cp = dataclasses.replace(cp, needs_layout_passes=False)
    mesh = plsc.VectorSubcoreMesh(core_axis_name="c", subcore_axis_name="s")
    grid_pw = pl.kernel(
        _sc_body,
        out_type=jax.ShapeDtypeStruct((_B, _P, _W), jnp.float32),
        mesh=mesh,
        scratch_types=[
            pltpu.VMEM((_SHARE,), jnp.int32),
            pltpu.VMEM((_NBMAX, _K), jnp.int32),
            pltpu.VMEM((_NBMAX, _K), jnp.int32),
            pltpu.VMEM((_K, _W), jnp.float32),
            pltpu.VMEM((_ZROWS, _W), jnp.float32),
            pltpu.VMEM_SHARED((_CH + _NS * 16, _W), jnp.float32),
        ],
        compiler_params=cp,
    )(x_pad, idx_pad, zrow)

    out = pl.pallas_call(
        _transpose_body,
        grid=(_B, _P // _T_BLK),
        in_specs=[pl.BlockSpec((1, _T_BLK, _W), lambda b, t: (b, t, 0))],
        out_specs=pl.BlockSpec((1, _C, _T_BLK), lambda b, t: (b, 0, t)),
        out_shape=jax.ShapeDtypeStruct((_B, _C, _P), jnp.float32),
    )(grid_pw)
    return out.reshape(_B, _C, _PX, _PY)
